# baseline (speedup 1.0000x reference)
import functools

import jax
import jax.numpy as jnp
from jax import lax
from jax.experimental import pallas as pl
from jax.experimental.pallas import tpu as pltpu
from jax.experimental.pallas import tpu_sc as plsc

N = 10000
D = 128
H = 128
ODE_TIME = 1.0

NP = 10240
NPT = NP // 16
NC = 2
NS = 16
L = 16


def _hw_body(x_ref, wint_ref, bin_ref, wgcnt_ref, out_ref):
    h = lax.dot_general(x_ref[...], wint_ref[...], (((1,), (0,)), ((), ())),
                        preferred_element_type=jnp.float32)
    h = jnp.maximum(h + bin_ref[...], 0.0)
    out_ref[...] = lax.dot_general(h, wgcnt_ref[...], (((1,), (0,)), ((), ())),
                                   preferred_element_type=jnp.float32)


def _compute_hw(x, W_inT, b_in2, W_gcnT):
    blk = 128
    grid = NP // blk
    return pl.pallas_call(
        _hw_body,
        grid=(grid,),
        in_specs=[
            pl.BlockSpec((blk, D), lambda b: (b, 0)),
            pl.BlockSpec((D, H), lambda b: (0, 0)),
            pl.BlockSpec((1, H), lambda b: (0, 0)),
            pl.BlockSpec((H, H), lambda b: (0, 0)),
        ],
        out_specs=pl.BlockSpec((blk, H), lambda b: (b, 0)),
        out_shape=jax.ShapeDtypeStruct((NP, H), jnp.float32),
    )(x, W_inT, b_in2, W_gcnT)


def _expm_body(w_ref, out_ref):
    a = w_ref[...] * (ODE_TIME / 16.0)
    eye = jnp.eye(H, dtype=jnp.float32)
    acc = eye + a
    term = a
    for k in range(2, 13):
        term = lax.dot_general(term, a, (((1,), (0,)), ((), ())),
                               preferred_element_type=jnp.float32,
                        precision=lax.Precision.HIGHEST) * (1.0 / k)
        acc = acc + term
    for _ in range(4):
        acc = lax.dot_general(acc, acc, (((1,), (0,)), ((), ())),
                              preferred_element_type=jnp.float32,
                        precision=lax.Precision.HIGHEST)
    out_ref[...] = acc


def _compute_expm(W_ode):
    return pl.pallas_call(
        _expm_body,
        out_shape=jax.ShapeDtypeStruct((H, H), jnp.float32),
    )(W_ode)


def _sc_body(rp_h, cp_h, ew_h, hw_h, z2_h, z1_h, s2_h,
             agg_s, deg_s, dinv_s,
             cblk, rblk, eblk, gbufA, gbufB, dinvb, dbuf, nbuf,
             gsem0, gsem1, ssem0, ssem1, dsem):
    c = lax.axis_index("c")
    s = lax.axis_index("s")
    g = c * NS + s
    nrows = cp_h.shape[0]
    deg_rows = nrows // NS
    agg_rows = nrows // (NC * NS)

    pltpu.sync_copy(z2_h, agg_s.at[pl.ds(s * NPT, NPT)])
    pltpu.sync_copy(z1_h.at[pl.ds(s * NPT, NPT)], deg_s.at[pl.ds(s * NPT, NPT)])
    plsc.subcore_barrier()

    def deg_outer(t, carry):
        base = s * deg_rows + t * 8
        pltpu.sync_copy(cp_h.at[pl.ds(base, 8)], cblk)
        pltpu.sync_copy(ew_h.at[pl.ds(base, 8)], eblk)
        descs = [pltpu.async_copy(eblk.at[j], deg_s.at[cblk.at[j]], dsem,
                                  add=True) for j in range(8)]
        for d in descs:
            d.wait()
        return carry

    lax.fori_loop(0, deg_rows // 8, deg_outer, 0)
    plsc.subcore_barrier()

    pltpu.sync_copy(deg_s.at[pl.ds(s * NPT, NPT)], dbuf)

    def rsqrt_body(k, carry):
        v = dbuf[pl.ds(k * L, L)]
        sq = (v + 1.0) * 0.5
        for _ in range(7):
            sq = (sq + v / sq) * 0.5
        dbuf[pl.ds(k * L, L)] = 1.0 / sq
        return carry

    lax.fori_loop(0, NPT // L, rsqrt_body, 0)
    pltpu.sync_copy(dbuf, dinv_s.at[pl.ds(s * NPT, NPT)])
    plsc.subcore_barrier()
    pltpu.sync_copy(dinv_s, dinvb)

    gbs = (gbufA, gbufB)
    gsems = (gsem0, gsem1)
    ssems = (ssem0, ssem1)

    r0 = max(128, (nrows * 23 // 100) // 128 * 128)
    r1 = nrows - r0
    rows_tile = jnp.where(c == 0, r0 // NS, r1 // NS)
    tile_base = jnp.where(c == 0, s * (r0 // NS), r0 + s * (r1 // NS))

    def agg_outer(t, carry):
        base = tile_base + t * 8
        pltpu.sync_copy(cp_h.at[pl.ds(base, 8)], cblk)
        pltpu.sync_copy(rp_h.at[pl.ds(base, 8)], rblk)
        pltpu.sync_copy(ew_h.at[pl.ds(base, 8)], eblk)

        pltpu.async_copy(hw_h.at[rblk.at[0]], gbs[0], gsems[0])
        for j in range(8):
            p = j % 2
            q = 1 - p
            gb = gbs[p]
            pltpu.make_async_copy(hw_h.at[rblk.at[j]], gb, gsems[p]).wait()
            if j + 1 < 8:
                if j >= 1:
                    pltpu.make_async_copy(
                        gbs[q], agg_s.at[cblk.at[j - 1]], ssems[q]).wait()
                pltpu.async_copy(hw_h.at[rblk.at[j + 1]], gbs[q], gsems[q])

            def norm_body(k, carry3, _j=j):
                rv = rblk[_j, pl.ds(k * L, L)]
                cv = cblk[_j, pl.ds(k * L, L)]
                ev = eblk[_j, pl.ds(k * L, L)]
                dr = plsc.load_gather(dinvb, [rv])
                dc = plsc.load_gather(dinvb, [cv])
                nbuf[pl.ds(k * L, L)] = dr * ev * dc
                return carry3

            lax.fori_loop(0, 128 // L, norm_body, 0)

            def row_body(m, carry3, _gb=gb):
                bm = plsc.load_gather(nbuf, [lax.broadcast(m, (L,))])
                for k in range(H // L):
                    _gb[m, pl.ds(k * L, L)] = _gb[m, pl.ds(k * L, L)] * bm
                return carry3

            lax.fori_loop(0, 128, row_body, 0)
            pltpu.async_copy(gb, agg_s.at[cblk.at[j]], ssems[p], add=True)

        pltpu.make_async_copy(gbs[0], agg_s.at[cblk.at[6]], ssems[0]).wait()
        pltpu.make_async_copy(gbs[1], agg_s.at[cblk.at[7]], ssems[1]).wait()
        return carry

    lax.fori_loop(0, rows_tile // 8, agg_outer, 0)
    plsc.subcore_barrier()

    pltpu.sync_copy(agg_s.at[pl.ds(s * NPT, NPT)],
                    s2_h.at[pl.ds(c * NP + s * NPT, NPT)])


def _compute_agg(rp2, cp2, ew2, hw, z2, z1):
    nrows = rp2.shape[0]
    body = functools.partial(_sc_body)
    return pl.kernel(
        body,
        out_type=jax.ShapeDtypeStruct((NC * NP, H), jnp.float32),
        mesh=plsc.VectorSubcoreMesh(core_axis_name="c", subcore_axis_name="s"),
        compiler_params=pltpu.CompilerParams(needs_layout_passes=False),
        scratch_types=[
            pltpu.VMEM_SHARED((NP, H), jnp.float32),
            pltpu.VMEM_SHARED((NP,), jnp.float32),
            pltpu.VMEM_SHARED((NP,), jnp.float32),
            pltpu.VMEM((8, 128), jnp.int32),
            pltpu.VMEM((8, 128), jnp.int32),
            pltpu.VMEM((8, 128), jnp.float32),
            pltpu.VMEM((128, H), jnp.float32),
            pltpu.VMEM((128, H), jnp.float32),
            pltpu.VMEM((NP,), jnp.float32),
            pltpu.VMEM((NPT,), jnp.float32),
            pltpu.VMEM((128,), jnp.float32),
            pltpu.SemaphoreType.DMA,
            pltpu.SemaphoreType.DMA,
            pltpu.SemaphoreType.DMA,
            pltpu.SemaphoreType.DMA,
            pltpu.SemaphoreType.DMA,
        ],
    )(rp2, cp2, ew2, hw, z2, z1)


def _out_body(s0_ref, s1_ref, mt_ref, wout_ref, bout_ref, o_ref):
    t = jnp.maximum(s0_ref[...] + s1_ref[...], 0.0)
    u = lax.dot_general(t, mt_ref[...], (((1,), (0,)), ((), ())),
                        preferred_element_type=jnp.float32)
    o_ref[...] = jnp.sum(u * wout_ref[...], axis=1) + bout_ref[0, 0]


def _compute_out(S2, MT, W_out, b_out2):
    blk = 128
    grid = NP // blk
    return pl.pallas_call(
        _out_body,
        grid=(grid,),
        in_specs=[
            pl.BlockSpec((blk, H), lambda b: (b, 0)),
            pl.BlockSpec((blk, H), lambda b, _g=grid: (b + _g, 0)),
            pl.BlockSpec((H, H), lambda b: (0, 0)),
            pl.BlockSpec((1, H), lambda b: (0, 0)),
            pl.BlockSpec(memory_space=pltpu.SMEM),
        ],
        out_specs=pl.BlockSpec((blk,), lambda b: (b,)),
        out_shape=jax.ShapeDtypeStruct((NP,), jnp.float32),
    )(S2, S2, MT, W_out, b_out2)


def kernel(x, edge_index, edge_weight, W_in, b_in, W_gcn, W_ode, W_out, b_out):
    E0 = edge_index.shape[1]
    nrows = -(-(E0 + N) // (128 * 256)) * 256
    EP = nrows * 128
    pad = EP - (E0 + N)

    loop = jnp.arange(N, dtype=jnp.int32)
    zi = jnp.zeros((pad,), jnp.int32)
    rp = jnp.concatenate([edge_index[0].astype(jnp.int32), loop, zi])
    cp = jnp.concatenate([edge_index[1].astype(jnp.int32), loop, zi])
    ew = jnp.concatenate([edge_weight.astype(jnp.float32),
                          jnp.ones((N,), jnp.float32),
                          jnp.zeros((pad,), jnp.float32)])
    rp2 = rp.reshape(nrows, 128)
    cp2 = cp.reshape(nrows, 128)
    ew2 = ew.reshape(nrows, 128)

    z2 = jnp.zeros((NPT, H), jnp.float32)
    z1 = jnp.zeros((NP,), jnp.float32)

    xp = jnp.concatenate([x, jnp.zeros((NP - N, D), jnp.float32)], axis=0)
    hw = _compute_hw(xp, W_in.T, b_in.reshape(1, H), W_gcn.T)
    M = _compute_expm(W_ode)
    S2 = _compute_agg(rp2, cp2, ew2, hw, z2, z1)
    out = _compute_out(S2, M.T, W_out, b_out.reshape(1, 1))
    return out[:N]


# speedup vs baseline: 1.1845x; 1.1845x over previous
import functools

import jax
import jax.numpy as jnp
from jax import lax
from jax.experimental import pallas as pl
from jax.experimental.pallas import tpu as pltpu
from jax.experimental.pallas import tpu_sc as plsc

N = 10000
D = 128
H = 128
ODE_TIME = 1.0

NP = 10240
NPT = NP // 16
NC = 2
NS = 16
L = 16


def _hw_body(x_ref, wint_ref, bin_ref, wgcnt_ref, out_ref):
    h = lax.dot_general(x_ref[...], wint_ref[...], (((1,), (0,)), ((), ())),
                        preferred_element_type=jnp.float32)
    h = jnp.maximum(h + bin_ref[...], 0.0)
    out_ref[...] = lax.dot_general(h, wgcnt_ref[...], (((1,), (0,)), ((), ())),
                                   preferred_element_type=jnp.float32)


def _compute_hw(x, W_inT, b_in2, W_gcnT):
    blk = 128
    grid = NP // blk
    return pl.pallas_call(
        _hw_body,
        grid=(grid,),
        in_specs=[
            pl.BlockSpec((blk, D), lambda b: (b, 0)),
            pl.BlockSpec((D, H), lambda b: (0, 0)),
            pl.BlockSpec((1, H), lambda b: (0, 0)),
            pl.BlockSpec((H, H), lambda b: (0, 0)),
        ],
        out_specs=pl.BlockSpec((blk, H), lambda b: (b, 0)),
        out_shape=jax.ShapeDtypeStruct((NP, H), jnp.float32),
    )(x, W_inT, b_in2, W_gcnT)


def _expm_body(w_ref, out_ref):
    a = w_ref[...] * (ODE_TIME / 16.0)
    eye = jnp.eye(H, dtype=jnp.float32)
    acc = eye + a
    term = a
    for k in range(2, 13):
        term = lax.dot_general(term, a, (((1,), (0,)), ((), ())),
                               preferred_element_type=jnp.float32,
                        precision=lax.Precision.HIGHEST) * (1.0 / k)
        acc = acc + term
    for _ in range(4):
        acc = lax.dot_general(acc, acc, (((1,), (0,)), ((), ())),
                              preferred_element_type=jnp.float32,
                        precision=lax.Precision.HIGHEST)
    out_ref[...] = acc


def _compute_expm(W_ode):
    return pl.pallas_call(
        _expm_body,
        out_shape=jax.ShapeDtypeStruct((H, H), jnp.float32),
    )(W_ode)


def _sc_body(rp_h, cp_h, ew_h, hw_h, z2_h, z1_h, s2_h, degp_h,
             agg_s, dinv_s,
             cblk, rblk, eblk, gbufA, gbufB, dinvb, dbuf, nbuf,
             gsem0, gsem1, ssem0, ssem1, dsem):
    c = lax.axis_index("c")
    s = lax.axis_index("s")
    g = c * NS + s
    nrows = cp_h.shape[0]
    deg_rows = nrows // NS
    agg_rows = nrows // (NC * NS)

    pltpu.sync_copy(z2_h, agg_s.at[pl.ds(s * NPT, NPT)])
    pltpu.sync_copy(z1_h, dinvb)
    plsc.subcore_barrier()

    def deg_outer(t, carry):
        base = s * deg_rows + t * 8
        pltpu.sync_copy(cp_h.at[pl.ds(base, 8)], cblk)
        pltpu.sync_copy(ew_h.at[pl.ds(base, 8)], eblk)
        for j in range(8):
            def deg_vec(k, carry2, _j=j):
                cv = cblk[_j, pl.ds(k * L, L)]
                ev = eblk[_j, pl.ds(k * L, L)]
                plsc.addupdate_scatter(dinvb, [cv], ev)
                return carry2
            lax.fori_loop(0, 128 // L, deg_vec, 0)
        return carry

    lax.fori_loop(0, deg_rows // 8, deg_outer, 0)
    pltpu.sync_copy(dinvb, degp_h.at[g])
    plsc.subcore_barrier()

    descs = [pltpu.async_copy(
        degp_h.at[c * NS + t, pl.ds(s * NPT, NPT)],
        dinvb.at[pl.ds(t * NPT, NPT)], dsem) for t in range(NS)]
    for d in descs:
        d.wait()

    def seg_sum(k, carry):
        acc = dinvb[pl.ds(k * L, L)]
        for t in range(1, NS):
            acc = acc + dinvb[pl.ds(t * NPT + k * L, L)]
        dbuf[pl.ds(k * L, L)] = acc
        return carry

    lax.fori_loop(0, NPT // L, seg_sum, 0)


    def rsqrt_body(k, carry):
        v = dbuf[pl.ds(k * L, L)]
        sq = (v + 1.0) * 0.5
        for _ in range(7):
            sq = (sq + v / sq) * 0.5
        dbuf[pl.ds(k * L, L)] = 1.0 / sq
        return carry

    lax.fori_loop(0, NPT // L, rsqrt_body, 0)
    pltpu.sync_copy(dbuf, dinv_s.at[pl.ds(s * NPT, NPT)])
    plsc.subcore_barrier()
    pltpu.sync_copy(dinv_s, dinvb)

    gbs = (gbufA, gbufB)
    gsems = (gsem0, gsem1)
    ssems = (ssem0, ssem1)

    def agg_outer(t, carry):
        base = g * agg_rows + t * 8
        pltpu.sync_copy(cp_h.at[pl.ds(base, 8)], cblk)
        pltpu.sync_copy(rp_h.at[pl.ds(base, 8)], rblk)
        pltpu.sync_copy(ew_h.at[pl.ds(base, 8)], eblk)

        pltpu.async_copy(hw_h.at[rblk.at[0]], gbs[0], gsems[0])
        for j in range(8):
            p = j % 2
            q = 1 - p
            gb = gbs[p]
            pltpu.make_async_copy(hw_h.at[rblk.at[j]], gb, gsems[p]).wait()
            if j + 1 < 8:
                if j >= 1:
                    pltpu.make_async_copy(
                        gbs[q], agg_s.at[cblk.at[j - 1]], ssems[q]).wait()
                pltpu.async_copy(hw_h.at[rblk.at[j + 1]], gbs[q], gsems[q])

            def norm_body(k, carry3, _j=j):
                rv = rblk[_j, pl.ds(k * L, L)]
                cv = cblk[_j, pl.ds(k * L, L)]
                ev = eblk[_j, pl.ds(k * L, L)]
                dr = plsc.load_gather(dinvb, [rv])
                dc = plsc.load_gather(dinvb, [cv])
                nbuf[pl.ds(k * L, L)] = dr * ev * dc
                return carry3

            lax.fori_loop(0, 128 // L, norm_body, 0)

            def row_body(m, carry3, _gb=gb):
                bm = plsc.load_gather(nbuf, [lax.broadcast(m, (L,))])
                for k in range(H // L):
                    _gb[m, pl.ds(k * L, L)] = _gb[m, pl.ds(k * L, L)] * bm
                return carry3

            lax.fori_loop(0, 128, row_body, 0)
            pltpu.async_copy(gb, agg_s.at[cblk.at[j]], ssems[p], add=True)

        pltpu.make_async_copy(gbs[0], agg_s.at[cblk.at[6]], ssems[0]).wait()
        pltpu.make_async_copy(gbs[1], agg_s.at[cblk.at[7]], ssems[1]).wait()
        return carry

    lax.fori_loop(0, agg_rows // 8, agg_outer, 0)
    plsc.subcore_barrier()

    pltpu.sync_copy(agg_s.at[pl.ds(s * NPT, NPT)],
                    s2_h.at[pl.ds(c * NP + s * NPT, NPT)])


def _compute_agg(rp2, cp2, ew2, hw, z2, z1):
    nrows = rp2.shape[0]
    body = functools.partial(_sc_body)
    return pl.kernel(
        body,
        out_type=(jax.ShapeDtypeStruct((NC * NP, H), jnp.float32),
                  jax.ShapeDtypeStruct((NC * NS, NP), jnp.float32)),
        mesh=plsc.VectorSubcoreMesh(core_axis_name="c", subcore_axis_name="s"),
        compiler_params=pltpu.CompilerParams(needs_layout_passes=False),
        scratch_types=[
            pltpu.VMEM_SHARED((NP, H), jnp.float32),
            pltpu.VMEM_SHARED((NP,), jnp.float32),
            pltpu.VMEM((8, 128), jnp.int32),
            pltpu.VMEM((8, 128), jnp.int32),
            pltpu.VMEM((8, 128), jnp.float32),
            pltpu.VMEM((128, H), jnp.float32),
            pltpu.VMEM((128, H), jnp.float32),
            pltpu.VMEM((NP,), jnp.float32),
            pltpu.VMEM((NPT,), jnp.float32),
            pltpu.VMEM((128,), jnp.float32),
            pltpu.SemaphoreType.DMA,
            pltpu.SemaphoreType.DMA,
            pltpu.SemaphoreType.DMA,
            pltpu.SemaphoreType.DMA,
            pltpu.SemaphoreType.DMA,
        ],
    )(rp2, cp2, ew2, hw, z2, z1)


def _out_body(s0_ref, s1_ref, mt_ref, wout_ref, bout_ref, o_ref):
    t = jnp.maximum(s0_ref[...] + s1_ref[...], 0.0)
    u = lax.dot_general(t, mt_ref[...], (((1,), (0,)), ((), ())),
                        preferred_element_type=jnp.float32)
    o_ref[...] = jnp.sum(u * wout_ref[...], axis=1) + bout_ref[0, 0]


def _compute_out(S2, MT, W_out, b_out2):
    blk = 128
    grid = NP // blk
    return pl.pallas_call(
        _out_body,
        grid=(grid,),
        in_specs=[
            pl.BlockSpec((blk, H), lambda b: (b, 0)),
            pl.BlockSpec((blk, H), lambda b, _g=grid: (b + _g, 0)),
            pl.BlockSpec((H, H), lambda b: (0, 0)),
            pl.BlockSpec((1, H), lambda b: (0, 0)),
            pl.BlockSpec(memory_space=pltpu.SMEM),
        ],
        out_specs=pl.BlockSpec((blk,), lambda b: (b,)),
        out_shape=jax.ShapeDtypeStruct((NP,), jnp.float32),
    )(S2, S2, MT, W_out, b_out2)


def kernel(x, edge_index, edge_weight, W_in, b_in, W_gcn, W_ode, W_out, b_out):
    E0 = edge_index.shape[1]
    nrows = -(-(E0 + N) // (128 * 256)) * 256
    EP = nrows * 128
    pad = EP - (E0 + N)

    loop = jnp.arange(N, dtype=jnp.int32)
    zi = jnp.zeros((pad,), jnp.int32)
    rp = jnp.concatenate([edge_index[0].astype(jnp.int32), loop, zi])
    cp = jnp.concatenate([edge_index[1].astype(jnp.int32), loop, zi])
    ew = jnp.concatenate([edge_weight.astype(jnp.float32),
                          jnp.ones((N,), jnp.float32),
                          jnp.zeros((pad,), jnp.float32)])
    rp2 = rp.reshape(nrows, 128)
    cp2 = cp.reshape(nrows, 128)
    ew2 = ew.reshape(nrows, 128)

    z2 = jnp.zeros((NPT, H), jnp.float32)
    z1 = jnp.zeros((NP,), jnp.float32)

    xp = jnp.concatenate([x, jnp.zeros((NP - N, D), jnp.float32)], axis=0)
    hw = _compute_hw(xp, W_in.T, b_in.reshape(1, H), W_gcn.T)
    M = _compute_expm(W_ode)
    S2, _ = _compute_agg(rp2, cp2, ew2, hw, z2, z1)
    out = _compute_out(S2, M.T, W_out, b_out.reshape(1, 1))
    return out[:N]
